# baseline (speedup 1.0000x reference)
import functools

import jax
import jax.numpy as jnp
from jax import lax
from jax.experimental import pallas as pl
from jax.experimental.pallas import tpu as pltpu
from jax.experimental.pallas import tpu_sc as plsc

VOCAB = 1000000
EMB = 64
B = 4096
L = 200
CLS = 128

NC = 2
NS = 16
NW = NC * NS
ROWS_PER_W = B // NW
CHUNK = 100
CHUNKS_PER_ROW = L // CHUNK
NCHUNK = ROWS_PER_W * CHUNKS_PER_ROW
NLANE = EMB // 16


def _pool_body(seq_hbm, table_hbm, out_hbm, idx_v, buf_v, out_v, sem):
    wid = lax.axis_index("s") * NC + lax.axis_index("c")
    base = wid * NCHUNK
    pltpu.sync_copy(seq_hbm.at[pl.ds(base, NCHUNK)], idx_v)

    def row_body(i, carry):
        cp0 = pltpu.async_copy(table_hbm.at[idx_v.at[2 * i]], buf_v.at[0], sem)
        cp1 = pltpu.async_copy(table_hbm.at[idx_v.at[2 * i + 1]], buf_v.at[1], sem)
        cp0.wait()
        cp1.wait()

        def max_body(j, acc):
            new = []
            for c in range(NLANE):
                m = acc[c]
                for half in range(CHUNKS_PER_ROW):
                    m = jnp.maximum(m, buf_v[half, j, pl.ds(c * 16, 16)])
                new.append(m)
            return tuple(new)

        init = tuple(jnp.full((16,), -jnp.inf, jnp.float32)
                     for _ in range(NLANE))
        acc = lax.fori_loop(0, CHUNK, max_body, init)
        for c in range(NLANE):
            out_v[i, pl.ds(c * 16, 16)] = acc[c]
        return carry

    lax.fori_loop(0, ROWS_PER_W, row_body, 0)
    pltpu.sync_copy(out_v, out_hbm.at[pl.ds(wid * ROWS_PER_W, ROWS_PER_W)])


@functools.partial(
    pl.kernel,
    out_type=jax.ShapeDtypeStruct((B, EMB), jnp.float32),
    mesh=plsc.VectorSubcoreMesh(core_axis_name="c", subcore_axis_name="s"),
    scratch_types=[
        pltpu.VMEM((NCHUNK, CHUNK), jnp.int32),
        pltpu.VMEM((CHUNKS_PER_ROW, CHUNK, EMB), jnp.float32),
        pltpu.VMEM((ROWS_PER_W, EMB), jnp.float32),
        pltpu.SemaphoreType.DMA,
    ],
    compiler_params=pltpu.CompilerParams(use_tc_tiling_on_sc=False),
)
def _pool_sc(seq_hbm, table_hbm, out_hbm, idx_v, buf_v, out_v, sem):
    _pool_body(seq_hbm, table_hbm, out_hbm, idx_v, buf_v, out_v, sem)


def _mlp_body(x_ref, w_ref, b_ref, o_ref):
    o_ref[...] = (
        jnp.dot(x_ref[...], w_ref[...], preferred_element_type=jnp.float32)
        + b_ref[...]
    )


def _mlp(x, w, b2):
    return pl.pallas_call(
        _mlp_body,
        out_shape=jax.ShapeDtypeStruct((B, CLS), jnp.float32),
    )(x, w, b2)


def kernel(tokenizedSeqArr, table, W, b):
    seq2 = tokenizedSeqArr.reshape(B * CHUNKS_PER_ROW, CHUNK)
    pooled = _pool_sc(seq2, table)
    return _mlp(pooled, W, b.reshape(1, CLS))


# speedup vs baseline: 1.1309x; 1.1309x over previous
import functools

import jax
import jax.numpy as jnp
from jax import lax
from jax.experimental import pallas as pl
from jax.experimental.pallas import tpu as pltpu
from jax.experimental.pallas import tpu_sc as plsc

VOCAB = 1000000
EMB = 64
B = 4096
L = 200
CLS = 128

NC = 2
NS = 16
NW = NC * NS
ROWS_PER_W = B // NW
CHUNK = 100
CHUNKS_PER_ROW = L // CHUNK
NCHUNK = ROWS_PER_W * CHUNKS_PER_ROW
NLANE = EMB // 16


def _pool_body(seq_hbm, table_hbm, out_hbm, idx_v, buf_v, out_v, sem0, sem1):
    wid = lax.axis_index("s") * NC + lax.axis_index("c")
    base = wid * NCHUNK
    pltpu.sync_copy(seq_hbm.at[pl.ds(base, NCHUNK)], idx_v)

    sems = (sem0, sem1)

    def issue(row, slot):
        sem = sems[slot]
        for half in range(CHUNKS_PER_ROW):
            pltpu.async_copy(
                table_hbm.at[idx_v.at[CHUNKS_PER_ROW * row + half]],
                buf_v.at[slot, half], sem)

    def drain(row, slot):
        sem = sems[slot]
        for half in range(CHUNKS_PER_ROW):
            pltpu.make_async_copy(
                table_hbm.at[idx_v.at[CHUNKS_PER_ROW * row + half]],
                buf_v.at[slot, half], sem).wait()

    def compute(i, slot):
        def max_body(j, acc):
            new = list(acc)
            for jj in range(2):
                for c in range(NLANE):
                    m = new[c]
                    for half in range(CHUNKS_PER_ROW):
                        m = jnp.maximum(
                            m, buf_v[slot, half, 2 * j + jj, pl.ds(c * 16, 16)])
                    new[c] = m
            return tuple(new)

        init = tuple(jnp.full((16,), -jnp.inf, jnp.float32)
                     for _ in range(NLANE))
        acc = lax.fori_loop(0, CHUNK // 2, max_body, init)
        for c in range(NLANE):
            out_v[i, pl.ds(c * 16, 16)] = acc[c]

    issue(0, 0)

    def pair_body(p, carry):
        i0 = 2 * p
        issue(i0 + 1, 1)
        drain(i0, 0)
        compute(i0, 0)

        @pl.when(i0 + 2 < ROWS_PER_W)
        def _():
            issue(i0 + 2, 0)

        drain(i0 + 1, 1)
        compute(i0 + 1, 1)
        return carry

    lax.fori_loop(0, ROWS_PER_W // 2, pair_body, 0)
    pltpu.sync_copy(out_v, out_hbm.at[pl.ds(wid * ROWS_PER_W, ROWS_PER_W)])


@functools.partial(
    pl.kernel,
    out_type=jax.ShapeDtypeStruct((B, EMB), jnp.float32),
    mesh=plsc.VectorSubcoreMesh(core_axis_name="c", subcore_axis_name="s"),
    scratch_types=[
        pltpu.VMEM((NCHUNK, CHUNK), jnp.int32),
        pltpu.VMEM((2, CHUNKS_PER_ROW, CHUNK, EMB), jnp.float32),
        pltpu.VMEM((ROWS_PER_W, EMB), jnp.float32),
        pltpu.SemaphoreType.DMA,
        pltpu.SemaphoreType.DMA,
    ],
    compiler_params=pltpu.CompilerParams(use_tc_tiling_on_sc=False),
)
def _pool_sc(seq_hbm, table_hbm, out_hbm, idx_v, buf_v, out_v, sem0, sem1):
    _pool_body(seq_hbm, table_hbm, out_hbm, idx_v, buf_v, out_v, sem0, sem1)


def _mlp_body(x_ref, w_ref, b_ref, o_ref):
    o_ref[...] = (
        jnp.dot(x_ref[...], w_ref[...], preferred_element_type=jnp.float32)
        + b_ref[...]
    )


def _mlp(x, w, b2):
    return pl.pallas_call(
        _mlp_body,
        out_shape=jax.ShapeDtypeStruct((B, CLS), jnp.float32),
    )(x, w, b2)


def kernel(tokenizedSeqArr, table, W, b):
    seq2 = tokenizedSeqArr.reshape(B * CHUNKS_PER_ROW, CHUNK)
    pooled = _pool_sc(seq2, table)
    return _mlp(pooled, W, b.reshape(1, CLS))


# speedup vs baseline: 1.1972x; 1.0586x over previous
import functools

import jax
import jax.numpy as jnp
from jax import lax
from jax.experimental import pallas as pl
from jax.experimental.pallas import tpu as pltpu
from jax.experimental.pallas import tpu_sc as plsc

VOCAB = 1000000
EMB = 64
B = 4096
L = 200
CLS = 128

NC = 2
NS = 16
NW = NC * NS
ROWS_PER_W = B // NW
CHUNK = 100
CHUNKS_PER_ROW = L // CHUNK
NCHUNK = ROWS_PER_W * CHUNKS_PER_ROW
NLANE = EMB // 16


NSLOT = 4
UNROLL = 4


def _pool_body(seq_hbm, table_hbm, out_hbm, idx_v, buf_v, out_v, sems):
    wid = lax.axis_index("s") * NC + lax.axis_index("c")
    base = wid * NCHUNK
    pltpu.sync_copy(seq_hbm.at[pl.ds(base, NCHUNK)], idx_v)

    def issue(row, slot):
        for half in range(CHUNKS_PER_ROW):
            pltpu.async_copy(
                table_hbm.at[idx_v.at[CHUNKS_PER_ROW * row + half]],
                buf_v.at[slot, half], sems[slot])

    def drain(row, slot):
        for half in range(CHUNKS_PER_ROW):
            pltpu.make_async_copy(
                table_hbm.at[idx_v.at[CHUNKS_PER_ROW * row + half]],
                buf_v.at[slot, half], sems[slot]).wait()

    def compute(i, slot):
        def max_body(j, acc):
            new = list(acc)
            for jj in range(UNROLL):
                for c in range(NLANE):
                    m = new[c]
                    for half in range(CHUNKS_PER_ROW):
                        m = jnp.maximum(
                            m,
                            buf_v[slot, half, UNROLL * j + jj,
                                  pl.ds(c * 16, 16)])
                    new[c] = m
            return tuple(new)

        init = tuple(jnp.full((16,), -jnp.inf, jnp.float32)
                     for _ in range(NLANE))
        acc = lax.fori_loop(0, CHUNK // UNROLL, max_body, init)
        for c in range(NLANE):
            out_v[i, pl.ds(c * 16, 16)] = acc[c]

    for s in range(NSLOT):
        issue(s, s)

    def group_body(g, carry):
        r0 = NSLOT * g
        for s in range(NSLOT):
            drain(r0 + s, s)
            compute(r0 + s, s)

            @pl.when(r0 + s + NSLOT < ROWS_PER_W)
            def _():
                issue(r0 + s + NSLOT, s)
        return carry

    lax.fori_loop(0, ROWS_PER_W // NSLOT, group_body, 0)
    pltpu.sync_copy(out_v, out_hbm.at[pl.ds(wid * ROWS_PER_W, ROWS_PER_W)])


@functools.partial(
    pl.kernel,
    out_type=jax.ShapeDtypeStruct((B, EMB), jnp.float32),
    mesh=plsc.VectorSubcoreMesh(core_axis_name="c", subcore_axis_name="s"),
    scratch_types=[
        pltpu.VMEM((NCHUNK, CHUNK), jnp.int32),
        pltpu.VMEM((NSLOT, CHUNKS_PER_ROW, CHUNK, EMB), jnp.float32),
        pltpu.VMEM((ROWS_PER_W, EMB), jnp.float32),
    ] + [pltpu.SemaphoreType.DMA] * NSLOT,
    compiler_params=pltpu.CompilerParams(use_tc_tiling_on_sc=False),
)
def _pool_sc(seq_hbm, table_hbm, out_hbm, idx_v, buf_v, out_v, *sems):
    _pool_body(seq_hbm, table_hbm, out_hbm, idx_v, buf_v, out_v, sems)


def _mlp_body(x_ref, w_ref, b_ref, o_ref):
    o_ref[...] = (
        jnp.dot(x_ref[...], w_ref[...], preferred_element_type=jnp.float32)
        + b_ref[...]
    )


def _mlp(x, w, b2):
    return pl.pallas_call(
        _mlp_body,
        out_shape=jax.ShapeDtypeStruct((B, CLS), jnp.float32),
    )(x, w, b2)


def kernel(tokenizedSeqArr, table, W, b):
    seq2 = tokenizedSeqArr.reshape(B * CHUNKS_PER_ROW, CHUNK)
    pooled = _pool_sc(seq2, table)
    return _mlp(pooled, W, b.reshape(1, CLS))
